# Initial kernel scaffold; baseline (speedup 1.0000x reference)
import jax
import jax.numpy as jnp
from jax.experimental import pallas as pl


def kernel(vecs):
    raise NotImplementedError("write your pallas kernel here")


# speedup vs baseline: 76.9952x; 76.9952x over previous
import numpy as np
import jax
import jax.numpy as jnp
from jax import lax
from jax.experimental import pallas as pl
from jax.experimental.pallas import tpu as pltpu

_GRID = 0.5
_N = 48
_VDW = 1.52
_MULTI = 1.5
_WEIGHT = 25.0
_B = _MULTI * _VDW


def _sq_threshold() -> np.float32:
    r = np.float32(_MULTI * _VDW)
    c = np.float32(r) * np.float32(r)
    while np.float32(np.sqrt(np.nextafter(c, np.float32(0.0), dtype=np.float32))) >= r:
        c = np.nextafter(c, np.float32(0.0), dtype=np.float32)
    while np.float32(np.sqrt(c)) < r:
        c = np.nextafter(c, np.float32(np.inf), dtype=np.float32)
    return c


_C = float(_sq_threshold())

_ABLK = 1024
_WR = 24
_WC = 896


def _splat_kernel(vec_ref, out_ref, acc_ref):
    i = pl.program_id(0)
    nblk = pl.num_programs(0)

    @pl.when(i == 0)
    def _():
        acc_ref[...] = jnp.zeros_like(acc_ref)

    oxh = 0.5 * lax.broadcasted_iota(jnp.int32, (_WR, _WC), 0).astype(jnp.float32)
    oyh = 0.5 * (lax.broadcasted_iota(jnp.int32, (_WR, _WC), 1) // 64).astype(jnp.float32)
    ozh = 0.5 * (lax.broadcasted_iota(jnp.int32, (_WR, _WC), 1) % 64).astype(jnp.float32)

    def body(a, _):
        vx = vec_ref[0, a]
        vy = vec_ref[1, a]
        vz = vec_ref[2, a]
        minx = jnp.maximum(0, ((vx - _B) / _GRID).astype(jnp.int32))
        miny = jnp.maximum(0, ((vy - _B) / _GRID).astype(jnp.int32))
        rb = (minx // 8) * 8
        yb = (miny // 2) * 2
        cb = (miny // 2) * 128
        dx = vx - (0.5 * rb.astype(jnp.float32) + oxh)
        dy = vy - (0.5 * yb.astype(jnp.float32) + oyh)
        dz = vz - ozh
        d2 = (dx * dx + dy * dy) + dz * dz
        val = jnp.where(d2 < _C, 1.0, 0.0).astype(jnp.float32)
        acc_ref[pl.ds(rb, _WR), pl.ds(cb, _WC)] += val
        return ()

    lax.fori_loop(0, _ABLK, body, (), unroll=False)

    @pl.when(i == nblk - 1)
    def _():
        covered = acc_ref[...] >= 0.9
        out_ref[0, :, :] = jnp.where(covered, 1.0, 0.0).astype(jnp.float32)
        out_ref[1, :, :] = jnp.where(covered, _WEIGHT, 1.0).astype(jnp.float32)


def kernel(vecs):
    n = vecs.shape[0]
    npad = ((n + _ABLK - 1) // _ABLK) * _ABLK
    vecs_t = jnp.full((3, npad), -1000.0, dtype=jnp.float32)
    vecs_t = vecs_t.at[:, :n].set(vecs.T)
    nblk = npad // _ABLK
    out = pl.pallas_call(
        _splat_kernel,
        grid=(nblk,),
        in_specs=[pl.BlockSpec((3, _ABLK), lambda i: (0, i), memory_space=pltpu.SMEM)],
        out_specs=pl.BlockSpec((2, 64, 4096), lambda i: (0, 0, 0)),
        out_shape=jax.ShapeDtypeStruct((2, 64, 4096), jnp.float32),
        scratch_shapes=[pltpu.VMEM((64, 4096), jnp.float32)],
    )(vecs_t)
    return out.reshape(2, 64, 64, 64)[:, :_N, :_N, :_N]
